# Initial kernel scaffold; baseline (speedup 1.0000x reference)
import jax
import jax.numpy as jnp
from jax.experimental import pallas as pl


def kernel(x, indices):
    raise NotImplementedError("write your pallas kernel here")


# speedup vs baseline: 76.7850x; 76.7850x over previous
import functools

import jax
import jax.numpy as jnp
from jax import lax
from jax.experimental import pallas as pl
from jax.experimental.pallas import tpu as pltpu
from jax.experimental.pallas import tpu_sc as plsc

B, C, Hp, Wp = 4, 96, 192, 192
H, W = 384, 384

NC, NS = 2, 16
NW = NC * NS

ROWS_PER_CHUNK = 32
IN_CH = ROWS_PER_CHUNK * Wp
OUT_CH = 2 * ROWS_PER_CHUNK * W
N_CHUNKS = (B * C * Hp) // ROWS_PER_CHUNK
CHUNKS_PER_TILE = N_CHUNKS // NW
SUBS_PER_PLANE = Hp // ROWS_PER_CHUNK


def _unpool_body(x_hbm, idx_hbm, out_hbm, xv, iv, buf):
    wid = lax.axis_index("c") * NS + lax.axis_index("s")
    zero = jnp.zeros((16,), jnp.float32)

    def chunk_body(t, _):
        c = wid * CHUNKS_PER_TILE + t
        in_base = c * IN_CH
        out_base = c * OUT_CH
        off = lax.rem(t, SUBS_PER_PLANE) * OUT_CH

        pltpu.sync_copy(x_hbm.at[pl.ds(in_base, IN_CH)], xv)
        pltpu.sync_copy(idx_hbm.at[pl.ds(in_base, IN_CH)], iv)

        def zbody(k, _):
            buf[pl.ds(k * 16, 16)] = zero
            return ()
        lax.fori_loop(0, OUT_CH // 16, zbody, (), unroll=8)

        def sbody(k, _):
            ivec = iv[pl.ds(k * 16, 16)] - off
            xvec = xv[pl.ds(k * 16, 16)]
            plsc.store_scatter(buf, [ivec], xvec)
            return ()
        lax.fori_loop(0, IN_CH // 16, sbody, (), unroll=8)

        pltpu.sync_copy(buf, out_hbm.at[pl.ds(out_base, OUT_CH)])
        return ()

    lax.fori_loop(0, CHUNKS_PER_TILE, chunk_body, ())


@jax.jit
def kernel(x, indices):
    xf = x.reshape(B * C * Hp * Wp)
    idxf = indices.reshape(B * C * Hp * Wp)
    mesh = plsc.VectorSubcoreMesh(core_axis_name="c", subcore_axis_name="s",
                                  num_cores=NC, num_subcores=NS)
    run = pl.kernel(
        _unpool_body,
        out_type=jax.ShapeDtypeStruct((B * C * H * W,), jnp.float32),
        mesh=mesh,
        scratch_types=[
            pltpu.VMEM((IN_CH,), jnp.float32),
            pltpu.VMEM((IN_CH,), jnp.int32),
            pltpu.VMEM((OUT_CH,), jnp.float32),
        ],
        compiler_params=pltpu.CompilerParams(needs_layout_passes=False),
    )
    out = run(xf, idxf)
    return out.reshape(B, C, H, W)


# speedup vs baseline: 90.9176x; 1.1841x over previous
import jax
import jax.numpy as jnp
from jax import lax
from jax.experimental import pallas as pl
from jax.experimental.pallas import tpu as pltpu
from jax.experimental.pallas import tpu_sc as plsc

B, C, Hp, Wp = 4, 96, 192, 192
H, W = 384, 384

NC, NS = 2, 16
NW = NC * NS

ROWS_PER_CHUNK = 32
IN_CH = ROWS_PER_CHUNK * Wp
OUT_CH = 2 * ROWS_PER_CHUNK * W
N_CHUNKS = (B * C * Hp) // ROWS_PER_CHUNK
CHUNKS_PER_TILE = N_CHUNKS // NW
SUBS_PER_PLANE = Hp // ROWS_PER_CHUNK
UNROLL = 8


def _unpool_body(x_hbm, idx_hbm, out_hbm,
                 xv0, xv1, xv2, iv0, iv1, iv2, buf0, buf1,
                 isem0, isem1, isem2, osem0, osem1):
    xvs = (xv0, xv1, xv2)
    ivs = (iv0, iv1, iv2)
    bufs = (buf0, buf1)
    isems = (isem0, isem1, isem2)
    osems = (osem0, osem1)

    wid = lax.axis_index("c") * NS + lax.axis_index("s")
    c_base = wid * CHUNKS_PER_TILE
    zero16 = jnp.zeros((16,), jnp.float32)

    def issue_in(c, s):
        pltpu.async_copy(x_hbm.at[pl.ds(c * IN_CH, IN_CH)], xvs[s], isems[s])
        pltpu.async_copy(idx_hbm.at[pl.ds(c * IN_CH, IN_CH)], ivs[s], isems[s])

    def wait_in(c, s):
        pltpu.make_async_copy(
            x_hbm.at[pl.ds(c * IN_CH, IN_CH)], xvs[s], isems[s]).wait()
        pltpu.make_async_copy(
            idx_hbm.at[pl.ds(c * IN_CH, IN_CH)], ivs[s], isems[s]).wait()

    def wait_out(c, bs):
        pltpu.make_async_copy(
            bufs[bs], out_hbm.at[pl.ds(c * OUT_CH, OUT_CH)], osems[bs]).wait()

    for j in range(3):
        issue_in(c_base + j, j)

    def outer(u, _):
        for t6 in range(6):
            t = u * 6 + t6
            c = c_base + t
            bs = t6 % 2
            ins = t6 % 3
            off = t6 * OUT_CH
            off_prev = ((t6 - 2) % 6) * OUT_CH

            @pl.when(t >= 2)
            def _():
                wait_out(c - 2, bs)

                def zs(k, _):
                    ivec = ivs[(t6 - 2) % 3][pl.ds(k * 16, 16)] - off_prev
                    plsc.store_scatter(bufs[bs], [ivec], zero16)
                    return ()
                lax.fori_loop(0, IN_CH // 16, zs, (), unroll=UNROLL)

            @pl.when(t < 2)
            def _():
                def zb(k, _):
                    bufs[bs][pl.ds(k * 16, 16)] = zero16
                    return ()
                lax.fori_loop(0, OUT_CH // 16, zb, (), unroll=UNROLL)

            @pl.when(jnp.logical_and(t >= 2, t < CHUNKS_PER_TILE - 1))
            def _():
                issue_in(c + 1, (t6 + 1) % 3)

            wait_in(c, ins)

            def sb(k, _):
                ivec = ivs[ins][pl.ds(k * 16, 16)] - off
                xvec = xvs[ins][pl.ds(k * 16, 16)]
                plsc.store_scatter(bufs[bs], [ivec], xvec)
                return ()
            lax.fori_loop(0, IN_CH // 16, sb, (), unroll=UNROLL)

            pltpu.async_copy(bufs[bs], out_hbm.at[pl.ds(c * OUT_CH, OUT_CH)],
                             osems[bs])
        return ()

    lax.fori_loop(0, CHUNKS_PER_TILE // 6, outer, ())

    for t in (CHUNKS_PER_TILE - 2, CHUNKS_PER_TILE - 1):
        wait_out(c_base + t, t % 2)


@jax.jit
def kernel(x, indices):
    xf = x.reshape(B * C * Hp * Wp)
    idxf = indices.reshape(B * C * Hp * Wp)
    mesh = plsc.VectorSubcoreMesh(core_axis_name="c", subcore_axis_name="s",
                                  num_cores=NC, num_subcores=NS)
    run = pl.kernel(
        _unpool_body,
        out_type=jax.ShapeDtypeStruct((B * C * H * W,), jnp.float32),
        mesh=mesh,
        scratch_types=[
            pltpu.VMEM((IN_CH,), jnp.float32),
            pltpu.VMEM((IN_CH,), jnp.float32),
            pltpu.VMEM((IN_CH,), jnp.float32),
            pltpu.VMEM((IN_CH,), jnp.int32),
            pltpu.VMEM((IN_CH,), jnp.int32),
            pltpu.VMEM((IN_CH,), jnp.int32),
            pltpu.VMEM((OUT_CH,), jnp.float32),
            pltpu.VMEM((OUT_CH,), jnp.float32),
            pltpu.SemaphoreType.DMA,
            pltpu.SemaphoreType.DMA,
            pltpu.SemaphoreType.DMA,
            pltpu.SemaphoreType.DMA,
            pltpu.SemaphoreType.DMA,
        ],
        compiler_params=pltpu.CompilerParams(needs_layout_passes=False),
    )
    out = run(xf, idxf)
    return out.reshape(B, C, H, W)


# speedup vs baseline: 128.3053x; 1.4112x over previous
import jax
import jax.numpy as jnp
from jax import lax
from jax.experimental import pallas as pl
from jax.experimental.pallas import tpu as pltpu
from jax.experimental.pallas import tpu_sc as plsc

B, C, Hp, Wp = 4, 96, 192, 192
H, W = 384, 384

NC, NS = 2, 16
NW = NC * NS

ROWS_PER_CHUNK = 32
IN_CH = ROWS_PER_CHUNK * Wp
OUT_CH = 2 * ROWS_PER_CHUNK * W
N_CHUNKS = (B * C * Hp) // ROWS_PER_CHUNK
CHUNKS_PER_TILE = N_CHUNKS // NW
SUBS_PER_PLANE = Hp // ROWS_PER_CHUNK
UNROLL = 8


def _unpool_body(x_hbm, idx_hbm, out_hbm,
                 xv0, xv1, xv2, iv0, iv1, iv2, buf0, buf1,
                 isem0, isem1, isem2, osem0, osem1):
    xvs = (xv0, xv1, xv2)
    ivs = (iv0, iv1, iv2)
    bufs = (buf0, buf1)
    isems = (isem0, isem1, isem2)
    osems = (osem0, osem1)

    wid = lax.axis_index("c") * NS + lax.axis_index("s")
    c_base = wid * CHUNKS_PER_TILE
    zero16 = jnp.zeros((16,), jnp.float32)

    def issue_in(c, s):
        pltpu.async_copy(x_hbm.at[pl.ds(c * IN_CH, IN_CH)], xvs[s], isems[s])
        pltpu.async_copy(idx_hbm.at[pl.ds(c * IN_CH, IN_CH)], ivs[s], isems[s])

    def wait_in(c, s):
        pltpu.make_async_copy(
            x_hbm.at[pl.ds(c * IN_CH, IN_CH)], xvs[s], isems[s]).wait()
        pltpu.make_async_copy(
            idx_hbm.at[pl.ds(c * IN_CH, IN_CH)], ivs[s], isems[s]).wait()

    def wait_out(c, bs):
        pltpu.make_async_copy(
            bufs[bs], out_hbm.at[pl.ds(c * OUT_CH, OUT_CH)], osems[bs]).wait()

    for j in range(3):
        issue_in(c_base + j, j)

    def outer(u, _):
        for t6 in range(6):
            t = u * 6 + t6
            c = c_base + t
            bs = t6 % 2
            ins = t6 % 3
            off = t6 * OUT_CH
            off_prev = ((t6 - 2) % 6) * OUT_CH

            @pl.when(t >= 2)
            def _():
                wait_out(c - 2, bs)

                @plsc.parallel_loop(0, IN_CH, 16, unroll=UNROLL)
                def zs(k):
                    ivec = ivs[(t6 - 2) % 3][pl.ds(k, 16)] - off_prev
                    plsc.store_scatter(bufs[bs], [ivec], zero16)

            @pl.when(t < 2)
            def _():
                @plsc.parallel_loop(0, OUT_CH, 16, unroll=UNROLL)
                def zb(k):
                    bufs[bs][pl.ds(k, 16)] = zero16

            @pl.when(jnp.logical_and(t >= 2, t < CHUNKS_PER_TILE - 1))
            def _():
                issue_in(c + 1, (t6 + 1) % 3)

            wait_in(c, ins)

            @plsc.parallel_loop(0, IN_CH, 16, unroll=UNROLL)
            def sb(k):
                ivec = ivs[ins][pl.ds(k, 16)] - off
                xvec = xvs[ins][pl.ds(k, 16)]
                plsc.store_scatter(bufs[bs], [ivec], xvec)

            pltpu.async_copy(bufs[bs], out_hbm.at[pl.ds(c * OUT_CH, OUT_CH)],
                             osems[bs])
        return ()

    lax.fori_loop(0, CHUNKS_PER_TILE // 6, outer, ())

    for t in (CHUNKS_PER_TILE - 2, CHUNKS_PER_TILE - 1):
        wait_out(c_base + t, t % 2)


@jax.jit
def kernel(x, indices):
    xf = x.reshape(B * C * Hp * Wp)
    idxf = indices.reshape(B * C * Hp * Wp)
    mesh = plsc.VectorSubcoreMesh(core_axis_name="c", subcore_axis_name="s",
                                  num_cores=NC, num_subcores=NS)
    run = pl.kernel(
        _unpool_body,
        out_type=jax.ShapeDtypeStruct((B * C * H * W,), jnp.float32),
        mesh=mesh,
        scratch_types=[
            pltpu.VMEM((IN_CH,), jnp.float32),
            pltpu.VMEM((IN_CH,), jnp.float32),
            pltpu.VMEM((IN_CH,), jnp.float32),
            pltpu.VMEM((IN_CH,), jnp.int32),
            pltpu.VMEM((IN_CH,), jnp.int32),
            pltpu.VMEM((IN_CH,), jnp.int32),
            pltpu.VMEM((OUT_CH,), jnp.float32),
            pltpu.VMEM((OUT_CH,), jnp.float32),
            pltpu.SemaphoreType.DMA,
            pltpu.SemaphoreType.DMA,
            pltpu.SemaphoreType.DMA,
            pltpu.SemaphoreType.DMA,
            pltpu.SemaphoreType.DMA,
        ],
        compiler_params=pltpu.CompilerParams(needs_layout_passes=False),
    )
    out = run(xf, idxf)
    return out.reshape(B, C, H, W)


# speedup vs baseline: 159.9746x; 1.2468x over previous
import jax
import jax.numpy as jnp
from jax import lax
from jax.experimental import pallas as pl
from jax.experimental.pallas import tpu as pltpu
from jax.experimental.pallas import tpu_sc as plsc

B, C, Hp, Wp = 4, 96, 192, 192
H, W = 384, 384

NC, NS = 2, 16
NW = NC * NS

ROWS_PER_CHUNK = 32
IN_CH = ROWS_PER_CHUNK * Wp
OUT_ROWS = 2 * ROWS_PER_CHUNK
OUT_CH = OUT_ROWS * W
N_CHUNKS = (B * C * Hp) // ROWS_PER_CHUNK
CHUNKS_PER_TILE = N_CHUNKS // NW
SUBS_PER_PLANE = Hp // ROWS_PER_CHUNK
PLANES_PER_TILE = CHUNKS_PER_TILE // SUBS_PER_PLANE
UNROLL = 2
VR = Wp // 16


def _unpool_body(x_hbm, idx_hbm, out_hbm,
                 xv0, xv1, xv2, iv0, iv1, iv2, buf0, buf1,
                 isem0, isem1, isem2, osem0, osem1):
    xvs = (xv0, xv1, xv2)
    ivs = (iv0, iv1, iv2)
    bufs = (buf0, buf1)
    isems = (isem0, isem1, isem2)
    osems = (osem0, osem1)

    wid = lax.axis_index("c") * NS + lax.axis_index("s")
    c_base = wid * CHUNKS_PER_TILE
    zero16 = jnp.zeros((16,), jnp.float32)

    def issue_in(c, s):
        pltpu.async_copy(x_hbm.at[pl.ds(c * IN_CH, IN_CH)], xvs[s], isems[s])
        pltpu.async_copy(idx_hbm.at[pl.ds(c * IN_CH, IN_CH)], ivs[s], isems[s])

    def wait_in(c, s):
        pltpu.make_async_copy(
            x_hbm.at[pl.ds(c * IN_CH, IN_CH)], xvs[s], isems[s]).wait()
        pltpu.make_async_copy(
            idx_hbm.at[pl.ds(c * IN_CH, IN_CH)], ivs[s], isems[s]).wait()

    def out_dst(plane, sub):
        b = lax.div(plane, C)
        ch = lax.rem(plane, C)
        return out_hbm.at[b, ch, pl.ds(sub * OUT_ROWS, OUT_ROWS), :]

    def scatter_chunk(iv, buf, off, vals_of):
        @plsc.parallel_loop(0, ROWS_PER_CHUNK, 1, unroll=UNROLL)
        def _(r):
            row_off = off + 768 * r
            for jj in range(VR):
                e = r * Wp + jj * 16
                ivec = iv[pl.ds(e, 16)] - row_off
                m = ivec >= W
                cvec = jnp.where(m, ivec - W, ivec)
                rvec = jnp.where(m, 2 * r + 1, 2 * r)
                plsc.store_scatter(buf, [rvec, cvec], vals_of(r, jj))

    for j in range(3):
        issue_in(c_base + j, j)

    def outer(u, _):
        plane = wid * PLANES_PER_TILE + u
        for t6 in range(SUBS_PER_PLANE):
            t = u * SUBS_PER_PLANE + t6
            c = c_base + t
            bs = t6 % 2
            ins = t6 % 3
            off = t6 * OUT_CH
            sub_prev = (t6 - 2) % SUBS_PER_PLANE
            off_prev = sub_prev * OUT_CH
            plane_prev = plane - 1 if t6 < 2 else plane

            @pl.when(t >= 2)
            def _():
                pltpu.make_async_copy(
                    bufs[bs], out_dst(plane_prev, sub_prev),
                    osems[bs]).wait()
                ivp = ivs[(t6 - 2) % 3]
                scatter_chunk(ivp, bufs[bs], off_prev,
                              lambda r, jj: zero16)

            @pl.when(t < 2)
            def _():
                @plsc.parallel_loop(0, OUT_ROWS, 1, unroll=UNROLL)
                def _(r):
                    for jj in range(W // 16):
                        bufs[bs][r, pl.ds(jj * 16, 16)] = zero16

            @pl.when(jnp.logical_and(t >= 2, t < CHUNKS_PER_TILE - 1))
            def _():
                issue_in(c + 1, (t6 + 1) % 3)

            wait_in(c, ins)
            scatter_chunk(ivs[ins], bufs[bs], off,
                          lambda r, jj: xvs[ins][pl.ds(r * Wp + jj * 16, 16)])

            pltpu.async_copy(bufs[bs], out_dst(plane, t6), osems[bs])
        return ()

    lax.fori_loop(0, PLANES_PER_TILE, outer, ())

    last_plane = wid * PLANES_PER_TILE + PLANES_PER_TILE - 1
    for t6 in (SUBS_PER_PLANE - 2, SUBS_PER_PLANE - 1):
        pltpu.make_async_copy(
            bufs[t6 % 2], out_dst(last_plane, t6), osems[t6 % 2]).wait()


@jax.jit
def kernel(x, indices):
    xf = x.reshape(B * C * Hp * Wp)
    idxf = indices.reshape(B * C * Hp * Wp)
    mesh = plsc.VectorSubcoreMesh(core_axis_name="c", subcore_axis_name="s",
                                  num_cores=NC, num_subcores=NS)
    run = pl.kernel(
        _unpool_body,
        out_type=jax.ShapeDtypeStruct((B, C, H, W), jnp.float32),
        mesh=mesh,
        scratch_types=[
            pltpu.VMEM((IN_CH,), jnp.float32),
            pltpu.VMEM((IN_CH,), jnp.float32),
            pltpu.VMEM((IN_CH,), jnp.float32),
            pltpu.VMEM((IN_CH,), jnp.int32),
            pltpu.VMEM((IN_CH,), jnp.int32),
            pltpu.VMEM((IN_CH,), jnp.int32),
            pltpu.VMEM((OUT_ROWS, W), jnp.float32),
            pltpu.VMEM((OUT_ROWS, W), jnp.float32),
            pltpu.SemaphoreType.DMA,
            pltpu.SemaphoreType.DMA,
            pltpu.SemaphoreType.DMA,
            pltpu.SemaphoreType.DMA,
            pltpu.SemaphoreType.DMA,
        ],
        compiler_params=pltpu.CompilerParams(needs_layout_passes=False),
    )
    return run(xf, idxf)


# speedup vs baseline: 250.6314x; 1.5667x over previous
import jax
import jax.numpy as jnp
from jax import lax
from jax.experimental import pallas as pl
from jax.experimental.pallas import tpu as pltpu
from jax.experimental.pallas import tpu_sc as plsc

B, C, Hp, Wp = 4, 96, 192, 192
H, W = 384, 384

NC, NS = 2, 16
NW = NC * NS

ROWS_PER_CHUNK = 32
IN_CH = ROWS_PER_CHUNK * Wp
OUT_ROWS = 2 * ROWS_PER_CHUNK
OUT_CH = OUT_ROWS * W
N_CHUNKS = (B * C * Hp) // ROWS_PER_CHUNK
CHUNKS_PER_TILE = N_CHUNKS // NW
SUBS_PER_PLANE = Hp // ROWS_PER_CHUNK
PLANES_PER_TILE = CHUNKS_PER_TILE // SUBS_PER_PLANE
UNROLL = 2
VR = Wp // 16


def _unpool_body(x_hbm, idx_hbm, out_hbm,
                 xv0, xv1, xv2, iv0, iv1, iv2, buf0, buf1,
                 isem0, isem1, isem2, osem0, osem1):
    xvs = (xv0, xv1, xv2)
    ivs = (iv0, iv1, iv2)
    bufs = (buf0, buf1)
    isems = (isem0, isem1, isem2)
    osems = (osem0, osem1)

    wid = lax.axis_index("c") * NS + lax.axis_index("s")
    zero16 = jnp.zeros((16,), jnp.float32)

    def bc(plane):
        return lax.div(plane, C), lax.rem(plane, C)

    def in_src(hbm, plane, sub):
        b, ch = bc(plane)
        return hbm.at[b, ch, pl.ds(sub * ROWS_PER_CHUNK, ROWS_PER_CHUNK), :]

    def issue_in(plane, sub, s):
        pltpu.async_copy(in_src(x_hbm, plane, sub), xvs[s], isems[s])
        pltpu.async_copy(in_src(idx_hbm, plane, sub), ivs[s], isems[s])

    def wait_in(plane, sub, s):
        pltpu.make_async_copy(in_src(x_hbm, plane, sub), xvs[s],
                              isems[s]).wait()
        pltpu.make_async_copy(in_src(idx_hbm, plane, sub), ivs[s],
                              isems[s]).wait()

    def out_dst(plane, sub):
        b, ch = bc(plane)
        return out_hbm.at[b, ch, pl.ds(sub * OUT_ROWS, OUT_ROWS), :]

    def scatter_chunk(iv, buf, off, vals_of):
        @plsc.parallel_loop(0, ROWS_PER_CHUNK, 1, unroll=UNROLL)
        def _(r):
            row_off = off + 768 * r
            for jj in range(VR):
                ivec = iv[r, pl.ds(jj * 16, 16)] - row_off
                m = ivec >= W
                cvec = jnp.where(m, ivec - W, ivec)
                rvec = jnp.where(m, 2 * r + 1, 2 * r)
                plsc.store_scatter(buf, [rvec, cvec], vals_of(r, jj))

    plane0 = wid * PLANES_PER_TILE
    for j in range(3):
        issue_in(plane0, j, j)

    def outer(u, _):
        plane = plane0 + u
        for t6 in range(SUBS_PER_PLANE):
            t = u * SUBS_PER_PLANE + t6
            bs = t6 % 2
            ins = t6 % 3
            off = t6 * OUT_CH
            sub_prev = (t6 - 2) % SUBS_PER_PLANE
            off_prev = sub_prev * OUT_CH
            plane_prev = plane - 1 if t6 < 2 else plane

            @pl.when(t >= 2)
            def _():
                pltpu.make_async_copy(
                    bufs[bs], out_dst(plane_prev, sub_prev), osems[bs]).wait()
                ivp = ivs[(t6 - 2) % 3]
                scatter_chunk(ivp, bufs[bs], off_prev, lambda r, jj: zero16)

            @pl.when(t < 2)
            def _():
                @plsc.parallel_loop(0, OUT_ROWS, 1, unroll=UNROLL)
                def _(r):
                    for jj in range(W // 16):
                        bufs[bs][r, pl.ds(jj * 16, 16)] = zero16

            @pl.when(jnp.logical_and(t >= 2, t < CHUNKS_PER_TILE - 1))
            def _():
                sub_next = (t6 + 1) % SUBS_PER_PLANE
                plane_next = plane + 1 if t6 == SUBS_PER_PLANE - 1 else plane
                issue_in(plane_next, sub_next, (t6 + 1) % 3)

            wait_in(plane, t6, ins)
            scatter_chunk(ivs[ins], bufs[bs], off,
                          lambda r, jj: xvs[ins][r, pl.ds(jj * 16, 16)])

            pltpu.async_copy(bufs[bs], out_dst(plane, t6), osems[bs])
        return ()

    lax.fori_loop(0, PLANES_PER_TILE, outer, ())

    last_plane = plane0 + PLANES_PER_TILE - 1
    for t6 in (SUBS_PER_PLANE - 2, SUBS_PER_PLANE - 1):
        pltpu.make_async_copy(
            bufs[t6 % 2], out_dst(last_plane, t6), osems[t6 % 2]).wait()


@jax.jit
def kernel(x, indices):
    mesh = plsc.VectorSubcoreMesh(core_axis_name="c", subcore_axis_name="s",
                                  num_cores=NC, num_subcores=NS)
    run = pl.kernel(
        _unpool_body,
        out_type=jax.ShapeDtypeStruct((B, C, H, W), jnp.float32),
        mesh=mesh,
        scratch_types=[
            pltpu.VMEM((ROWS_PER_CHUNK, Wp), jnp.float32),
            pltpu.VMEM((ROWS_PER_CHUNK, Wp), jnp.float32),
            pltpu.VMEM((ROWS_PER_CHUNK, Wp), jnp.float32),
            pltpu.VMEM((ROWS_PER_CHUNK, Wp), jnp.int32),
            pltpu.VMEM((ROWS_PER_CHUNK, Wp), jnp.int32),
            pltpu.VMEM((ROWS_PER_CHUNK, Wp), jnp.int32),
            pltpu.VMEM((OUT_ROWS, W), jnp.float32),
            pltpu.VMEM((OUT_ROWS, W), jnp.float32),
            pltpu.SemaphoreType.DMA,
            pltpu.SemaphoreType.DMA,
            pltpu.SemaphoreType.DMA,
            pltpu.SemaphoreType.DMA,
            pltpu.SemaphoreType.DMA,
        ],
        compiler_params=pltpu.CompilerParams(needs_layout_passes=False),
    )
    return run(x, indices)
